# baseline (speedup 1.0000x reference)
import functools

import jax
import jax.numpy as jnp
from jax import lax
from jax.experimental import pallas as pl
from jax.experimental.pallas import tpu as pltpu
from jax.experimental.pallas import tpu_sc as plsc

_SEQ = 2048
_UNITS = 32
_NC = 2
_NS = 16
_NW = _NC * _NS
_ROWS = _SEQ // _NW
_ROW_F = _SEQ * _UNITS
_WIN_D = _SEQ + _ROWS
_WIN_F = _WIN_D * _UNITS
_FIRE = 8


def _sc_body(emb_hbm, out_hbm, emb_v, win_v, *sems):
    cid = lax.axis_index("c")
    sid = lax.axis_index("s")
    wid = sid * _NC + cid

    pltpu.sync_copy(emb_hbm, emb_v)
    halves = [(emb_v[v, pl.ds(0, 16)], emb_v[v, pl.ds(16, 16)]) for v in range(5)]

    t1 = 62 + 64 * wid

    def fill_run(lo, hi, h):
        def body(ld, c):
            win_v[ld, pl.ds(0, 16)] = h[0]
            win_v[ld, pl.ds(16, 16)] = h[1]
            return c
        lax.fori_loop(lo, hi, body, 0)

    fill_run(0, t1, halves[0])
    for k in range(3):
        win_v[t1 + k, pl.ds(0, 16)] = halves[k + 1][0]
        win_v[t1 + k, pl.ds(16, 16)] = halves[k + 1][1]
    fill_run(t1 + 3, _WIN_D, halves[4])

    row0 = wid * _ROWS
    copies = [None] * _ROWS
    for r in range(_ROWS):
        if r >= _FIRE:
            copies[r - _FIRE].wait()
        src = win_v.at[pl.ds(_ROWS - 1 - r, _SEQ), :]
        dst = out_hbm.at[pl.ds((row0 + r) * _SEQ, _SEQ), :]
        copies[r] = pltpu.async_copy(src, dst, sems[r % _FIRE])
    for r in range(_ROWS - _FIRE, _ROWS):
        copies[r].wait()


_rel_pos_sc = functools.partial(
    pl.kernel,
    out_type=jax.ShapeDtypeStruct((_SEQ * _SEQ, _UNITS), jnp.float32),
    mesh=plsc.VectorSubcoreMesh(core_axis_name="c", subcore_axis_name="s"),
    compiler_params=pltpu.CompilerParams(use_tc_tiling_on_sc=False),
    scratch_types=(
        [pltpu.VMEM((2 * 2 + 1, _UNITS), jnp.float32),
         pltpu.VMEM((_WIN_D, _UNITS), jnp.float32)]
        + [pltpu.SemaphoreType.DMA] * _FIRE
    ),
)(_sc_body)


def kernel(embedding):
    flat = _rel_pos_sc(embedding)
    return flat.reshape(_SEQ, _SEQ, _UNITS)


# speedup vs baseline: 8.0924x; 8.0924x over previous
import functools

import jax
import jax.numpy as jnp
from jax.experimental import pallas as pl
from jax.experimental.pallas import tpu as pltpu

_SEQ = 2048
_UNITS = 32
_BI = 8


def _tc_body(p_ref, out_ref):
    bi = pl.program_id(0)
    i = bi * _BI + jax.lax.broadcasted_iota(jnp.int32, (_BI, 1, _SEQ), 0)
    j = jax.lax.broadcasted_iota(jnp.int32, (_BI, 1, _SEQ), 2)
    d = j - i
    p = [p_ref[v][None] for v in range(5)]
    out_ref[...] = jnp.where(
        d <= -2, p[0],
        jnp.where(d == -1, p[1],
                  jnp.where(d == 0, p[2],
                            jnp.where(d == 1, p[3], p[4]))))


def kernel(embedding):
    planes = jnp.broadcast_to(embedding[:, :, None], (5, _UNITS, _SEQ))
    out_t = pl.pallas_call(
        _tc_body,
        grid=(_SEQ // _BI,),
        in_specs=[pl.BlockSpec((5, _UNITS, _SEQ), lambda i: (0, 0, 0))],
        out_specs=pl.BlockSpec((_BI, _UNITS, _SEQ), lambda i: (i, 0, 0)),
        out_shape=jax.ShapeDtypeStruct((_SEQ, _UNITS, _SEQ), jnp.float32),
    )(planes)
    return out_t.transpose(0, 2, 1)


# speedup vs baseline: 9.7087x; 1.1997x over previous
import functools

import jax
import jax.numpy as jnp
from jax.experimental import pallas as pl
from jax.experimental.pallas import tpu as pltpu

_SEQ = 2048
_UNITS = 32
_BI = 16


def _tc_body(p_ref, out_ref):
    bi = pl.program_id(0)
    i = bi * _BI + jax.lax.broadcasted_iota(jnp.int32, (_BI, 1, _SEQ), 0)
    j = jax.lax.broadcasted_iota(jnp.int32, (_BI, 1, _SEQ), 2)
    d = j - i
    p = [p_ref[v][None] for v in range(5)]
    out_ref[...] = jnp.where(
        d <= -2, p[0],
        jnp.where(d == -1, p[1],
                  jnp.where(d == 0, p[2],
                            jnp.where(d == 1, p[3], p[4]))))


def kernel(embedding):
    planes = jnp.broadcast_to(embedding[:, :, None], (5, _UNITS, _SEQ))
    out_t = pl.pallas_call(
        _tc_body,
        grid=(_SEQ // _BI,),
        in_specs=[pl.BlockSpec((5, _UNITS, _SEQ), lambda i: (0, 0, 0))],
        out_specs=pl.BlockSpec((_BI, _UNITS, _SEQ), lambda i: (i, 0, 0)),
        out_shape=jax.ShapeDtypeStruct((_SEQ, _UNITS, _SEQ), jnp.float32),
    )(planes)
    return out_t.transpose(0, 2, 1)


# speedup vs baseline: 10.6100x; 1.0928x over previous
import functools

import jax
import jax.numpy as jnp
from jax.experimental import pallas as pl
from jax.experimental.pallas import tpu as pltpu

_SEQ = 2048
_UNITS = 32
_BI = 32


def _tc_body(p_ref, out_ref):
    bi = pl.program_id(0)
    i = bi * _BI + jax.lax.broadcasted_iota(jnp.int32, (_BI, 1, _SEQ), 0)
    j = jax.lax.broadcasted_iota(jnp.int32, (_BI, 1, _SEQ), 2)
    d = j - i
    p = [p_ref[v][None] for v in range(5)]
    out_ref[...] = jnp.where(
        d <= -2, p[0],
        jnp.where(d == -1, p[1],
                  jnp.where(d == 0, p[2],
                            jnp.where(d == 1, p[3], p[4]))))


def kernel(embedding):
    planes = jnp.broadcast_to(embedding[:, :, None], (5, _UNITS, _SEQ))
    out_t = pl.pallas_call(
        _tc_body,
        grid=(_SEQ // _BI,),
        in_specs=[pl.BlockSpec((5, _UNITS, _SEQ), lambda i: (0, 0, 0))],
        out_specs=pl.BlockSpec((_BI, _UNITS, _SEQ), lambda i: (i, 0, 0)),
        out_shape=jax.ShapeDtypeStruct((_SEQ, _UNITS, _SEQ), jnp.float32),
    )(planes)
    return out_t.transpose(0, 2, 1)


# speedup vs baseline: 10.9386x; 1.0310x over previous
import functools

import jax
import jax.numpy as jnp
from jax.experimental import pallas as pl
from jax.experimental.pallas import tpu as pltpu

_SEQ = 2048
_UNITS = 32
_BI = 64


def _tc_body(p_ref, out_ref):
    bi = pl.program_id(0)
    i = bi * _BI + jax.lax.broadcasted_iota(jnp.int32, (_BI, 1, _SEQ), 0)
    j = jax.lax.broadcasted_iota(jnp.int32, (_BI, 1, _SEQ), 2)
    d = j - i
    p = [p_ref[v][None] for v in range(5)]
    out_ref[...] = jnp.where(
        d <= -2, p[0],
        jnp.where(d == -1, p[1],
                  jnp.where(d == 0, p[2],
                            jnp.where(d == 1, p[3], p[4]))))


def kernel(embedding):
    planes = jnp.broadcast_to(embedding[:, :, None], (5, _UNITS, _SEQ))
    out_t = pl.pallas_call(
        _tc_body,
        grid=(_SEQ // _BI,),
        in_specs=[pl.BlockSpec((5, _UNITS, _SEQ), lambda i: (0, 0, 0))],
        out_specs=pl.BlockSpec((_BI, _UNITS, _SEQ), lambda i: (i, 0, 0)),
        out_shape=jax.ShapeDtypeStruct((_SEQ, _UNITS, _SEQ), jnp.float32),
    )(planes)
    return out_t.transpose(0, 2, 1)
